# baseline (speedup 1.0000x reference)
import functools

import jax
import jax.numpy as jnp
from jax.experimental import pallas as pl
from jax.experimental.pallas import tpu as pltpu

_CHUNK = 131072
_LANES = 128
_BIG = 2**30


def _maxpass(x_ref, rows, jpg, base_j, m0, w0, masked, n_cols, chunk_base):
    def body(j, carry):
        m, w = carry
        x = x_ref[:, pl.ds(j * _LANES, _LANES)]
        if masked:
            col = chunk_base + j * _LANES + jax.lax.broadcasted_iota(
                jnp.int32, (rows, _LANES), 1
            )
            x = jnp.where(col < n_cols, x, -jnp.inf)
        imp = x > m
        w = jnp.where(imp, base_j + j, w)
        m = jnp.maximum(m, x)
        return m, w

    return jax.lax.fori_loop(0, jpg, body, (m0, w0), unroll=4)


def _sumpass(x_ref, rows, jpg, m, masked, n_cols, chunk_base):
    zero = jnp.zeros((rows, _LANES), jnp.float32)

    def body(j, carry):
        s, t = carry
        x = x_ref[:, pl.ds(j * _LANES, _LANES)]
        if masked:
            col = chunk_base + j * _LANES + jax.lax.broadcasted_iota(
                jnp.int32, (rows, _LANES), 1
            )
            x = jnp.where(col < n_cols, x, -jnp.inf)
        e = jnp.exp(x - m)
        xe = x * e
        if masked:
            xe = jnp.where(col < n_cols, xe, 0.0)
        return s + e, t + xe

    return jax.lax.fori_loop(0, jpg, body, (zero, zero), unroll=4)


def _chunk_update(x_ref, i, rows, jpg, n_cols, masked, m_ref, s_ref, t_ref, w_ref):
    chunk_base = i * _CHUNK
    m_old = m_ref[...]
    m_new, w_new = _maxpass(
        x_ref, rows, jpg, i * jpg, m_old, w_ref[...], masked, n_cols, chunk_base
    )
    w_ref[...] = w_new
    m_ref[...] = m_new
    s_c, t_c = _sumpass(x_ref, rows, jpg, m_new, masked, n_cols, chunk_base)
    alpha = jnp.exp(m_old - m_new)
    s_ref[...] = s_ref[...] * alpha + s_c
    t_ref[...] = t_ref[...] * alpha + t_c


def _body(n_cols, n_chunks, x_ref, samp_ref, ent_ref, m_ref, s_ref, t_ref, w_ref):
    i = pl.program_id(0)
    rows = x_ref.shape[0]
    jpg = _CHUNK // _LANES

    @pl.when(i == 0)
    def _init():
        m_ref[...] = jnp.full((rows, _LANES), -jnp.inf, jnp.float32)
        s_ref[...] = jnp.zeros((rows, _LANES), jnp.float32)
        t_ref[...] = jnp.zeros((rows, _LANES), jnp.float32)
        w_ref[...] = jnp.zeros((rows, _LANES), jnp.int32)

    @pl.when(i < n_chunks - 1)
    def _main():
        _chunk_update(x_ref, i, rows, jpg, n_cols, False, m_ref, s_ref, t_ref, w_ref)

    @pl.when(i == n_chunks - 1)
    def _last():
        _chunk_update(x_ref, i, rows, jpg, n_cols, True, m_ref, s_ref, t_ref, w_ref)

        m_lane = m_ref[...]
        big_m = jnp.max(m_lane, axis=1, keepdims=True)
        a_f = jnp.exp(m_lane - big_m)
        s = jnp.sum(s_ref[...] * a_f, axis=1, keepdims=True)
        t = jnp.sum(t_ref[...] * a_f, axis=1, keepdims=True)
        ent_ref[...] = (big_m + jnp.log(s)) - t / s
        lane = jax.lax.broadcasted_iota(jnp.int32, (rows, _LANES), 1)
        idx = w_ref[...] * _LANES + lane
        cand = jnp.where(m_lane == big_m, idx, _BIG)
        samp_ref[...] = jnp.min(cand, axis=1, keepdims=True)


def kernel(logits):
    rows, n_cols = logits.shape
    n_chunks = pl.cdiv(n_cols, _CHUNK)
    samp, ent = pl.pallas_call(
        functools.partial(_body, n_cols, n_chunks),
        grid=(n_chunks,),
        in_specs=[pl.BlockSpec((rows, _CHUNK), lambda i: (0, i))],
        out_specs=[
            pl.BlockSpec((rows, 1), lambda i: (0, 0)),
            pl.BlockSpec((rows, 1), lambda i: (0, 0)),
        ],
        out_shape=[
            jax.ShapeDtypeStruct((rows, 1), jnp.int32),
            jax.ShapeDtypeStruct((rows, 1), jnp.float32),
        ],
        scratch_shapes=[
            pltpu.VMEM((rows, _LANES), jnp.float32),
            pltpu.VMEM((rows, _LANES), jnp.float32),
            pltpu.VMEM((rows, _LANES), jnp.float32),
            pltpu.VMEM((rows, _LANES), jnp.int32),
        ],
    )(logits)
    return (samp.reshape(rows), logits, ent.reshape(rows))


# speedup vs baseline: 1.1171x; 1.1171x over previous
import functools

import jax
import jax.numpy as jnp
from jax.experimental import pallas as pl
from jax.experimental.pallas import tpu as pltpu

_CHUNK = 65536
_LANES = 128
_BIG = 2**30


def _maxpass(x_ref, rows, jpg, base_j, m0, w0, masked, n_cols, chunk_base):
    def body(j, carry):
        m, w = carry
        x = x_ref[:, pl.ds(j * _LANES, _LANES)]
        if masked:
            col = chunk_base + j * _LANES + jax.lax.broadcasted_iota(
                jnp.int32, (rows, _LANES), 1
            )
            x = jnp.where(col < n_cols, x, -jnp.inf)
        imp = x > m
        w = jnp.where(imp, base_j + j, w)
        m = jnp.maximum(m, x)
        return m, w

    return jax.lax.fori_loop(0, jpg, body, (m0, w0), unroll=8)


def _sumpass(x_ref, rows, jpg, m, masked, n_cols, chunk_base):
    zero = jnp.zeros((rows, _LANES), jnp.float32)

    def body(j, carry):
        s, t = carry
        x = x_ref[:, pl.ds(j * _LANES, _LANES)]
        if masked:
            col = chunk_base + j * _LANES + jax.lax.broadcasted_iota(
                jnp.int32, (rows, _LANES), 1
            )
            x = jnp.where(col < n_cols, x, -jnp.inf)
        e = jnp.exp(x - m)
        xe = x * e
        if masked:
            xe = jnp.where(col < n_cols, xe, 0.0)
        return s + e, t + xe

    return jax.lax.fori_loop(0, jpg, body, (zero, zero), unroll=8)


def _chunk_update(x_ref, i, rows, jpg, n_cols, masked, m_ref, s_ref, t_ref, w_ref):
    chunk_base = i * _CHUNK
    m_old = m_ref[...]
    m_new, w_new = _maxpass(
        x_ref, rows, jpg, i * jpg, m_old, w_ref[...], masked, n_cols, chunk_base
    )
    w_ref[...] = w_new
    m_ref[...] = m_new
    s_c, t_c = _sumpass(x_ref, rows, jpg, m_new, masked, n_cols, chunk_base)
    alpha = jnp.exp(m_old - m_new)
    s_ref[...] = s_ref[...] * alpha + s_c
    t_ref[...] = t_ref[...] * alpha + t_c


def _body(n_cols, n_chunks, x_ref, samp_ref, ent_ref, m_ref, s_ref, t_ref, w_ref):
    i = pl.program_id(0)
    rows = x_ref.shape[0]
    jpg = _CHUNK // _LANES

    @pl.when(i == 0)
    def _init():
        m_ref[...] = jnp.full((rows, _LANES), -jnp.inf, jnp.float32)
        s_ref[...] = jnp.zeros((rows, _LANES), jnp.float32)
        t_ref[...] = jnp.zeros((rows, _LANES), jnp.float32)
        w_ref[...] = jnp.zeros((rows, _LANES), jnp.int32)

    @pl.when(i < n_chunks - 1)
    def _main():
        _chunk_update(x_ref, i, rows, jpg, n_cols, False, m_ref, s_ref, t_ref, w_ref)

    @pl.when(i == n_chunks - 1)
    def _last():
        _chunk_update(x_ref, i, rows, jpg, n_cols, True, m_ref, s_ref, t_ref, w_ref)

        m_lane = m_ref[...]
        big_m = jnp.max(m_lane, axis=1, keepdims=True)
        a_f = jnp.exp(m_lane - big_m)
        s = jnp.sum(s_ref[...] * a_f, axis=1, keepdims=True)
        t = jnp.sum(t_ref[...] * a_f, axis=1, keepdims=True)
        ent_ref[...] = (big_m + jnp.log(s)) - t / s
        lane = jax.lax.broadcasted_iota(jnp.int32, (rows, _LANES), 1)
        idx = w_ref[...] * _LANES + lane
        cand = jnp.where(m_lane == big_m, idx, _BIG)
        samp_ref[...] = jnp.min(cand, axis=1, keepdims=True)


def kernel(logits):
    rows, n_cols = logits.shape
    n_chunks = pl.cdiv(n_cols, _CHUNK)
    samp, ent = pl.pallas_call(
        functools.partial(_body, n_cols, n_chunks),
        grid=(n_chunks,),
        in_specs=[pl.BlockSpec((rows, _CHUNK), lambda i: (0, i))],
        out_specs=[
            pl.BlockSpec((rows, 1), lambda i: (0, 0)),
            pl.BlockSpec((rows, 1), lambda i: (0, 0)),
        ],
        out_shape=[
            jax.ShapeDtypeStruct((rows, 1), jnp.int32),
            jax.ShapeDtypeStruct((rows, 1), jnp.float32),
        ],
        scratch_shapes=[
            pltpu.VMEM((rows, _LANES), jnp.float32),
            pltpu.VMEM((rows, _LANES), jnp.float32),
            pltpu.VMEM((rows, _LANES), jnp.float32),
            pltpu.VMEM((rows, _LANES), jnp.int32),
        ],
    )(logits)
    return (samp.reshape(rows), logits, ent.reshape(rows))


# speedup vs baseline: 1.1551x; 1.0340x over previous
import functools

import jax
import jax.numpy as jnp
from jax.experimental import pallas as pl
from jax.experimental.pallas import tpu as pltpu

_CHUNK = 65536
_LANES = 128
_BIG = 2**30


def _maxpass(x_ref, rows, jpg, base_j, m0, w0, masked, n_cols, chunk_base):
    def body(j, carry):
        m, w = carry
        x = x_ref[:, pl.ds(j * _LANES, _LANES)]
        if masked:
            col = chunk_base + j * _LANES + jax.lax.broadcasted_iota(
                jnp.int32, (rows, _LANES), 1
            )
            x = jnp.where(col < n_cols, x, -jnp.inf)
        imp = x > m
        w = jnp.where(imp, base_j + j, w)
        m = jnp.maximum(m, x)
        return m, w

    return jax.lax.fori_loop(0, jpg, body, (m0, w0), unroll=16)


def _sumpass(x_ref, rows, jpg, m, masked, n_cols, chunk_base):
    zero = jnp.zeros((rows, _LANES), jnp.float32)

    def body(j, carry):
        s, t = carry
        x = x_ref[:, pl.ds(j * _LANES, _LANES)]
        if masked:
            col = chunk_base + j * _LANES + jax.lax.broadcasted_iota(
                jnp.int32, (rows, _LANES), 1
            )
            x = jnp.where(col < n_cols, x, -jnp.inf)
        e = jnp.exp(x - m)
        xe = x * e
        if masked:
            xe = jnp.where(col < n_cols, xe, 0.0)
        return s + e, t + xe

    return jax.lax.fori_loop(0, jpg, body, (zero, zero), unroll=16)


def _chunk_update(x_ref, i, rows, jpg, n_cols, masked, m_ref, s_ref, t_ref, w_ref):
    chunk_base = i * _CHUNK
    m_old = m_ref[...]
    m_new, w_new = _maxpass(
        x_ref, rows, jpg, i * jpg, m_old, w_ref[...], masked, n_cols, chunk_base
    )
    w_ref[...] = w_new
    m_ref[...] = m_new
    s_c, t_c = _sumpass(x_ref, rows, jpg, m_new, masked, n_cols, chunk_base)
    alpha = jnp.exp(m_old - m_new)
    s_ref[...] = s_ref[...] * alpha + s_c
    t_ref[...] = t_ref[...] * alpha + t_c


def _body(n_cols, n_chunks, x_ref, samp_ref, ent_ref, m_ref, s_ref, t_ref, w_ref):
    i = pl.program_id(0)
    rows = x_ref.shape[0]
    jpg = _CHUNK // _LANES

    @pl.when(i == 0)
    def _init():
        m_ref[...] = jnp.full((rows, _LANES), -jnp.inf, jnp.float32)
        s_ref[...] = jnp.zeros((rows, _LANES), jnp.float32)
        t_ref[...] = jnp.zeros((rows, _LANES), jnp.float32)
        w_ref[...] = jnp.zeros((rows, _LANES), jnp.int32)

    @pl.when(i < n_chunks - 1)
    def _main():
        _chunk_update(x_ref, i, rows, jpg, n_cols, False, m_ref, s_ref, t_ref, w_ref)

    @pl.when(i == n_chunks - 1)
    def _last():
        _chunk_update(x_ref, i, rows, jpg, n_cols, True, m_ref, s_ref, t_ref, w_ref)

        m_lane = m_ref[...]
        big_m = jnp.max(m_lane, axis=1, keepdims=True)
        a_f = jnp.exp(m_lane - big_m)
        s = jnp.sum(s_ref[...] * a_f, axis=1, keepdims=True)
        t = jnp.sum(t_ref[...] * a_f, axis=1, keepdims=True)
        ent_ref[...] = (big_m + jnp.log(s)) - t / s
        lane = jax.lax.broadcasted_iota(jnp.int32, (rows, _LANES), 1)
        idx = w_ref[...] * _LANES + lane
        cand = jnp.where(m_lane == big_m, idx, _BIG)
        samp_ref[...] = jnp.min(cand, axis=1, keepdims=True)


def kernel(logits):
    rows, n_cols = logits.shape
    n_chunks = pl.cdiv(n_cols, _CHUNK)
    samp, ent = pl.pallas_call(
        functools.partial(_body, n_cols, n_chunks),
        grid=(n_chunks,),
        in_specs=[pl.BlockSpec((rows, _CHUNK), lambda i: (0, i))],
        out_specs=[
            pl.BlockSpec((rows, 1), lambda i: (0, 0)),
            pl.BlockSpec((rows, 1), lambda i: (0, 0)),
        ],
        out_shape=[
            jax.ShapeDtypeStruct((rows, 1), jnp.int32),
            jax.ShapeDtypeStruct((rows, 1), jnp.float32),
        ],
        scratch_shapes=[
            pltpu.VMEM((rows, _LANES), jnp.float32),
            pltpu.VMEM((rows, _LANES), jnp.float32),
            pltpu.VMEM((rows, _LANES), jnp.float32),
            pltpu.VMEM((rows, _LANES), jnp.int32),
        ],
    )(logits)
    return (samp.reshape(rows), logits, ent.reshape(rows))


# speedup vs baseline: 1.1752x; 1.0174x over previous
import functools

import jax
import jax.numpy as jnp
from jax.experimental import pallas as pl
from jax.experimental.pallas import tpu as pltpu

_CHUNK = 65536
_LANES = 128
_BIG = 2**30


def _maxpass(x_ref, rows, jpg, base_j, m0, w0, masked, n_cols, chunk_base):
    def body(j, carry):
        m, w = carry
        x = x_ref[:, pl.ds(j * _LANES, _LANES)]
        if masked:
            col = chunk_base + j * _LANES + jax.lax.broadcasted_iota(
                jnp.int32, (rows, _LANES), 1
            )
            x = jnp.where(col < n_cols, x, -jnp.inf)
        imp = x > m
        w = jnp.where(imp, base_j + j, w)
        m = jnp.maximum(m, x)
        return m, w

    return jax.lax.fori_loop(0, jpg, body, (m0, w0), unroll=32)


def _sumpass(x_ref, rows, jpg, m, masked, n_cols, chunk_base):
    zero = jnp.zeros((rows, _LANES), jnp.float32)

    def body(j, carry):
        s, t = carry
        x = x_ref[:, pl.ds(j * _LANES, _LANES)]
        if masked:
            col = chunk_base + j * _LANES + jax.lax.broadcasted_iota(
                jnp.int32, (rows, _LANES), 1
            )
            x = jnp.where(col < n_cols, x, -jnp.inf)
        e = jnp.exp(x - m)
        xe = x * e
        if masked:
            xe = jnp.where(col < n_cols, xe, 0.0)
        return s + e, t + xe

    return jax.lax.fori_loop(0, jpg, body, (zero, zero), unroll=32)


def _chunk_update(x_ref, i, rows, jpg, n_cols, masked, m_ref, s_ref, t_ref, w_ref):
    chunk_base = i * _CHUNK
    m_old = m_ref[...]
    m_new, w_new = _maxpass(
        x_ref, rows, jpg, i * jpg, m_old, w_ref[...], masked, n_cols, chunk_base
    )
    w_ref[...] = w_new
    m_ref[...] = m_new
    s_c, t_c = _sumpass(x_ref, rows, jpg, m_new, masked, n_cols, chunk_base)
    alpha = jnp.exp(m_old - m_new)
    s_ref[...] = s_ref[...] * alpha + s_c
    t_ref[...] = t_ref[...] * alpha + t_c


def _body(n_cols, n_chunks, x_ref, samp_ref, ent_ref, m_ref, s_ref, t_ref, w_ref):
    i = pl.program_id(0)
    rows = x_ref.shape[0]
    jpg = _CHUNK // _LANES

    @pl.when(i == 0)
    def _init():
        m_ref[...] = jnp.full((rows, _LANES), -jnp.inf, jnp.float32)
        s_ref[...] = jnp.zeros((rows, _LANES), jnp.float32)
        t_ref[...] = jnp.zeros((rows, _LANES), jnp.float32)
        w_ref[...] = jnp.zeros((rows, _LANES), jnp.int32)

    @pl.when(i < n_chunks - 1)
    def _main():
        _chunk_update(x_ref, i, rows, jpg, n_cols, False, m_ref, s_ref, t_ref, w_ref)

    @pl.when(i == n_chunks - 1)
    def _last():
        _chunk_update(x_ref, i, rows, jpg, n_cols, True, m_ref, s_ref, t_ref, w_ref)

        m_lane = m_ref[...]
        big_m = jnp.max(m_lane, axis=1, keepdims=True)
        a_f = jnp.exp(m_lane - big_m)
        s = jnp.sum(s_ref[...] * a_f, axis=1, keepdims=True)
        t = jnp.sum(t_ref[...] * a_f, axis=1, keepdims=True)
        ent_ref[...] = (big_m + jnp.log(s)) - t / s
        lane = jax.lax.broadcasted_iota(jnp.int32, (rows, _LANES), 1)
        idx = w_ref[...] * _LANES + lane
        cand = jnp.where(m_lane == big_m, idx, _BIG)
        samp_ref[...] = jnp.min(cand, axis=1, keepdims=True)


def kernel(logits):
    rows, n_cols = logits.shape
    n_chunks = pl.cdiv(n_cols, _CHUNK)
    samp, ent = pl.pallas_call(
        functools.partial(_body, n_cols, n_chunks),
        grid=(n_chunks,),
        in_specs=[pl.BlockSpec((rows, _CHUNK), lambda i: (0, i))],
        out_specs=[
            pl.BlockSpec((rows, 1), lambda i: (0, 0)),
            pl.BlockSpec((rows, 1), lambda i: (0, 0)),
        ],
        out_shape=[
            jax.ShapeDtypeStruct((rows, 1), jnp.int32),
            jax.ShapeDtypeStruct((rows, 1), jnp.float32),
        ],
        scratch_shapes=[
            pltpu.VMEM((rows, _LANES), jnp.float32),
            pltpu.VMEM((rows, _LANES), jnp.float32),
            pltpu.VMEM((rows, _LANES), jnp.float32),
            pltpu.VMEM((rows, _LANES), jnp.int32),
        ],
    )(logits)
    return (samp.reshape(rows), logits, ent.reshape(rows))
